# baseline (speedup 1.0000x reference)
import functools

import jax
import jax.numpy as jnp
from jax import lax
from jax.experimental import pallas as pl
from jax.experimental.pallas import tpu as pltpu
from jax.experimental.pallas import tpu_sc as plsc

NBITS = 100
DIM = 64
G = 6
NG = 17
TROWS = NG * 64 - 48
NC = 2
NS = 16
NW = NC * NS
LANES = 16


def _table_body(w_ref, t_ref):
    w = w_ref[...]
    diff = w[:, 1, :] - w[:, 0, :]
    basev = jnp.sum(w[:, 0, :], axis=0)
    r = lax.broadcasted_iota(jnp.int32, (TROWS, NBITS), 0)
    j = lax.broadcasted_iota(jnp.int32, (TROWS, NBITS), 1)
    grp = r // 64
    m = r % 64
    sel = (j // G == grp) & (((m >> (j % G)) & 1) == 1)
    mat = sel.astype(jnp.float32)
    t = lax.dot_general(mat, diff, (((1,), (0,)), ((), ())),
                        preferred_element_type=jnp.float32)
    is_last = (r[:, :1] >= (NG - 1) * 64).astype(jnp.float32)
    t_ref[...] = t + is_last * basev[None, :]


def _build_table(w3):
    return pl.pallas_call(
        _table_body,
        out_shape=jax.ShapeDtypeStruct((TROWS, DIM), jnp.float32),
    )(w3)


def _sc_lookup(t_flat, msg_flat, n_batch):
    bpw = n_batch // NW
    half = bpw // 2
    nbt = half // LANES

    mesh = plsc.VectorSubcoreMesh(core_axis_name="c", subcore_axis_name="s")

    @functools.partial(
        pl.kernel,
        out_type=jax.ShapeDtypeStruct((n_batch * DIM,), jnp.float32),
        mesh=mesh,
        compiler_params=pltpu.CompilerParams(needs_layout_passes=False),
        scratch_types=[
            pltpu.VMEM((TROWS * DIM,), jnp.float32),
            pltpu.VMEM((half * NBITS,), jnp.int32),
            pltpu.VMEM((bpw * DIM,), jnp.float32),
        ],
    )
    def sc_kernel(t_hbm, msg_hbm, out_hbm, t_v, msg_v, out_v):
        cid = lax.axis_index("c")
        sid = lax.axis_index("s")
        wid = sid * NC + cid
        row0 = wid * bpw

        pltpu.sync_copy(t_hbm, t_v)
        li = lax.iota(jnp.int32, LANES)

        for h in range(2):
            pltpu.sync_copy(
                msg_hbm.at[pl.ds((row0 + h * half) * NBITS, half * NBITS)],
                msg_v)

            def btile(bt, _, h=h):
                ibase = (bt * LANES + li) * NBITS
                obase = ((h * half + bt * LANES) + li) * DIM
                rbs = []
                for g in range(NG):
                    nb = G if g < NG - 1 else NBITS - G * (NG - 1)
                    m = plsc.load_gather(msg_v, [ibase + G * g])
                    for i in range(1, nb):
                        bit = plsc.load_gather(msg_v, [ibase + (G * g + i)])
                        m = m + (bit << i)
                    rbs.append(g * (64 * DIM) + m * DIM)

                def cchunk(cc, _):
                    for c in range(LANES):
                        col = cc * LANES + c
                        acc = plsc.load_gather(t_v, [rbs[0] + col])
                        for g in range(1, NG):
                            acc = acc + plsc.load_gather(t_v, [rbs[g] + col])
                        plsc.store_scatter(out_v, [obase + col], acc)
                    return 0

                lax.fori_loop(0, DIM // LANES, cchunk, 0)
                return 0

            lax.fori_loop(0, nbt, btile, 0)

        pltpu.sync_copy(out_v, out_hbm.at[pl.ds(row0 * DIM, bpw * DIM)])

    return sc_kernel(t_flat, msg_flat)


def kernel(msg, emb_weight):
    n_batch, n_bits = msg.shape
    w3 = emb_weight.reshape(n_bits, 2, DIM)
    t = _build_table(w3)
    out = _sc_lookup(t.reshape(-1), msg.reshape(-1), n_batch)
    return out.reshape(n_batch, DIM)


# speedup vs baseline: 3.3698x; 3.3698x over previous
import functools

import jax
import jax.numpy as jnp
from jax import lax
from jax.experimental import pallas as pl
from jax.experimental.pallas import tpu as pltpu
from jax.experimental.pallas import tpu_sc as plsc

NBITS = 100
DIM = 64
G = 6
NG = 17
TROWS = NG * 64 - 48
NC = 2
NS = 16
NW = NC * NS
LANES = 16


def _table_body(w_ref, t_ref):
    w = w_ref[...]
    diff = w[:, 1, :] - w[:, 0, :]
    basev = jnp.sum(w[:, 0, :], axis=0)
    r = lax.broadcasted_iota(jnp.int32, (TROWS, NBITS), 0)
    j = lax.broadcasted_iota(jnp.int32, (TROWS, NBITS), 1)
    grp = r // 64
    m = r % 64
    sel = (j // G == grp) & (((m >> (j % G)) & 1) == 1)
    mat = sel.astype(jnp.float32)
    t = lax.dot_general(mat, diff, (((1,), (0,)), ((), ())),
                        preferred_element_type=jnp.float32)
    is_last = (r[:, :1] >= (NG - 1) * 64).astype(jnp.float32)
    t_ref[...] = t + is_last * basev[None, :]


def _build_table(w3):
    return pl.pallas_call(
        _table_body,
        out_shape=jax.ShapeDtypeStruct((TROWS, DIM), jnp.float32),
    )(w3)


def _sc_lookup(t_flat, msg_flat, n_batch):
    bpw = n_batch // NW
    half = bpw // 2
    nbt = half // LANES

    mesh = plsc.VectorSubcoreMesh(core_axis_name="c", subcore_axis_name="s")

    @functools.partial(
        pl.kernel,
        out_type=jax.ShapeDtypeStruct((n_batch * DIM,), jnp.float32),
        mesh=mesh,
        compiler_params=pltpu.CompilerParams(needs_layout_passes=False),
        scratch_types=[
            pltpu.VMEM((TROWS * DIM,), jnp.float32),
            pltpu.VMEM((half * NBITS,), jnp.int32),
            pltpu.VMEM((bpw * DIM,), jnp.float32),
        ],
    )
    def sc_kernel(t_hbm, msg_hbm, out_hbm, t_v, msg_v, out_v):
        cid = lax.axis_index("c")
        sid = lax.axis_index("s")
        wid = sid * NC + cid
        row0 = wid * bpw

        pltpu.sync_copy(t_hbm, t_v)
        li = lax.iota(jnp.int32, LANES)

        for h in range(2):
            pltpu.sync_copy(
                msg_hbm.at[pl.ds((row0 + h * half) * NBITS, half * NBITS)],
                msg_v)

            def btile(bt, _, h=h):
                ibase = (bt * LANES + li) * NBITS
                obase = ((h * half + bt * LANES) + li) * DIM
                rbs = []
                for g in range(NG):
                    nb = G if g < NG - 1 else NBITS - G * (NG - 1)
                    m = plsc.load_gather(msg_v, [ibase + G * g])
                    for i in range(1, nb):
                        bit = plsc.load_gather(msg_v, [ibase + (G * g + i)])
                        m = m + (bit << i)
                    rbs.append(g * (64 * DIM) + m * DIM)

                def cchunk(cc, _):
                    for k in range(LANES):
                        col = cc * LANES + ((li + k) & (LANES - 1))
                        acc = plsc.load_gather(t_v, [rbs[0] + col])
                        for g in range(1, NG):
                            acc = acc + plsc.load_gather(t_v, [rbs[g] + col])
                        plsc.store_scatter(out_v, [obase + col], acc)
                    return 0

                lax.fori_loop(0, DIM // LANES, cchunk, 0)
                return 0

            lax.fori_loop(0, nbt, btile, 0)

        pltpu.sync_copy(out_v, out_hbm.at[pl.ds(row0 * DIM, bpw * DIM)])

    return sc_kernel(t_flat, msg_flat)


def kernel(msg, emb_weight):
    n_batch, n_bits = msg.shape
    w3 = emb_weight.reshape(n_bits, 2, DIM)
    t = _build_table(w3)
    out = _sc_lookup(t.reshape(-1), msg.reshape(-1), n_batch)
    return out.reshape(n_batch, DIM)
